# baseline (speedup 1.0000x reference)
import functools

import jax
import jax.numpy as jnp
from jax.experimental import pallas as pl
from jax.experimental.pallas import tpu as pltpu

_B, _N, _E = 256, 64, 256
_D, _DE, _DS, _NF, _NH, _DH, _OUT = 128, 64, 8, 10, 4, 32, 512
_BB = 16
_STEPS = _B // _BB
_F32 = jnp.float32


def _silu(x):
    return x * jax.nn.sigmoid(x)


def _ln(x, g, b):
    m = jnp.mean(x, axis=-1, keepdims=True)
    v = jnp.mean((x - m) ** 2, axis=-1, keepdims=True)
    return (x - m) / jnp.sqrt(v + 1e-5) * g + b


def _dot(a, b):
    return jnp.dot(a, b, preferred_element_type=_F32)


def _doth(a, b):
    return jnp.dot(a, b, preferred_element_type=_F32,
                   precision=jax.lax.Precision.HIGHEST)


def _forward(xyz, xyznm, fo, jo, consts, interpret=False):
    names = list(consts)

    def body(xyz_r, xyznm_r, fo_r, jo_r, *refs):
        cr = dict(zip(names, refs[: len(names)]))
        out_r = refs[len(names)]

        def g(n):
            return cr[n][...]

        Gij, GiT, Gd = g("Gij"), g("GiT"), g("Gd")
        rest = g("rest")

        diff = _dot(Gd, xyznm_r[0])
        d2A = _dot(diff * diff, g("S3"))
        distA = jnp.sqrt(d2A + 1e-9)
        dlA = (distA - rest) / (rest + 1e-9)
        d2cols = [d2A[:, b:b + 1] for b in range(_BB)]
        dlcols = [dlA[:, b:b + 1] for b in range(_BB)]

        xyzf = xyz_r[...].reshape(_BB * _N, 3)
        ff = fo_r[...].reshape(_BB * _N, 5)
        jj = jo_r[...].reshape(_BB * _N, 8)
        xb = _doth(xyzf, g("P"))
        h = (_doth(xyzf, g("Wx")) + _dot(jnp.sin(xb), g("Ws"))
             + _dot(jnp.cos(xb), g("Wc")) + _doth(ff, g("Wf"))
             + _doth(jj, g("Wjn")) + g("b1"))
        h = _ln(h, g("ln1g"), g("ln1b"))
        h = 0.5 * h * (1.0 + jax.lax.erf(h * (2.0 ** -0.5)))
        h = _dot(h, g("W2")) + g("b2")
        H = _ln(h, g("ln2g"), g("ln2b"))

        for l in range(3):
            p = lambda n: g(f"L{l}_{n}")
            HA = _dot(H, p("Whi"))
            HB = _dot(H, p("Whj"))
            m1s, ggs = [], []
            for b in range(_BB):
                hAB = jnp.concatenate(
                    [HA[b * _N:(b + 1) * _N], HB[b * _N:(b + 1) * _N]],
                    axis=0)
                m1s.append(_dot(Gij, hAB)
                           + d2cols[b] * p("wd2") + dlcols[b] * p("wdl")
                           + p("geoC"))
                gt = _silu(d2cols[b] * p("wgd2") + dlcols[b] * p("wgdl")
                           + p("gateC"))
                ggs.append(jax.nn.sigmoid(
                    jnp.sum(gt * p("wg2r"), axis=-1, keepdims=True)
                    + p("bg2")))
            m1 = jnp.concatenate(m1s, axis=0)
            m2 = _silu(_dot(_silu(m1), p("We2")) + p("be2"))
            aggs = [_dot(GiT, m2[b * _E:(b + 1) * _E] * ggs[b])
                    for b in range(_BB)]
            agg = jnp.concatenate(aggs, axis=0)
            upd = _silu(_dot(H, p("Wn1h")) + _dot(agg, p("Wn1a")) + p("bn1"))
            upd = _dot(upd, p("Wn2")) + p("bn2")
            H = _ln(H + upd, p("lng"), p("lnb"))

        q = g("qc")
        k = _dot(H, g("Wk")) + g("bk")
        v = _dot(H, g("Wv")) + g("bv")
        sb = _doth(k * q, g("Mh")).reshape(_BB, _N, _NH)
        mx = jnp.max(sb, axis=1, keepdims=True)
        e = jnp.exp(sb - mx)
        att = (e / jnp.sum(e, axis=1, keepdims=True)).reshape(_BB * _N, _NH)
        ae = _doth(att, g("MhT"))
        z = jnp.sum((ae * v).reshape(_BB, _N, _D), axis=1)
        z = _doth(z, g("Wo")) + g("bo")
        z = _ln(z, g("plng"), g("plnb"))
        y = _silu(_doth(z, g("Wo1")) + g("bo1"))
        out_r[...] = _doth(y, g("Wo2")) + g("bo2")

    def cmap(r):
        return lambda b: (0,) * r

    in_specs = [
        pl.BlockSpec((_BB, _N, 3), lambda b: (b, 0, 0)),
        pl.BlockSpec((1, _N, _BB * 3), lambda b: (b, 0, 0)),
        pl.BlockSpec((_BB, _N, 5), lambda b: (b, 0, 0)),
        pl.BlockSpec((_BB, _N, 8), lambda b: (b, 0, 0)),
    ] + [pl.BlockSpec(consts[n].shape, cmap(consts[n].ndim)) for n in names]

    return pl.pallas_call(
        body,
        grid=(_STEPS,),
        in_specs=in_specs,
        out_specs=pl.BlockSpec((_BB, _OUT), lambda b: (b, 0)),
        out_shape=jax.ShapeDtypeStruct((_B, _OUT), _F32),
        compiler_params=pltpu.CompilerParams(
            dimension_semantics=("parallel",)),
        interpret=interpret,
    )(xyz, xyznm, fo, jo, *consts.values())


def _prepare(xyz, finger_ids, joint_type_ids, edge_index, edge_type,
             edge_rest_lengths, params):
    i, j = edge_index[0], edge_index[1]
    an = jnp.arange(_N)
    Gi = (i[:, None] == an).astype(_F32)
    Gj = (j[:, None] == an).astype(_F32)
    fo = (finger_ids[..., None] == jnp.arange(5)).astype(_F32)
    jo = (joint_type_ids[..., None] == jnp.arange(8)).astype(_F32)
    struct = (edge_type[:, None] == jnp.arange(4)).astype(_F32) \
        @ params["edge_struct_emb"]

    C = {}
    C["Gij"] = jnp.concatenate([Gi, Gj], axis=1)
    C["GiT"], C["Gd"] = Gi.T, Gi - Gj
    C["rest"] = edge_rest_lengths.reshape(_E, 1)
    bidx = jnp.arange(_BB * 3) // 3
    C["S3"] = (bidx[:, None] == jnp.arange(_BB)).astype(_F32)
    P = jnp.zeros((3, 3 * _NF), _F32)
    freqs = 2.0 ** jnp.arange(_NF, dtype=_F32)
    for c in range(3):
        P = P.at[c, c * _NF:(c + 1) * _NF].set(freqs)
    C["P"] = P
    W1, b1 = params["proj1"]["W"], params["proj1"]["b"]
    C["Wx"], C["Ws"], C["Wc"] = W1[0:3], W1[3:33], W1[33:63]
    C["Wf"] = params["finger_emb"] @ W1[63:79]
    C["Wjn"] = params["joint_emb"] @ W1[79:95]
    C["b1"] = b1.reshape(1, -1)
    C["ln1g"] = params["proj_ln1"]["g"].reshape(1, -1)
    C["ln1b"] = params["proj_ln1"]["b"].reshape(1, -1)
    C["W2"] = params["proj2"]["W"]
    C["b2"] = params["proj2"]["b"].reshape(1, -1)
    C["ln2g"] = params["proj_ln2"]["g"].reshape(1, -1)
    C["ln2b"] = params["proj_ln2"]["b"].reshape(1, -1)
    for l, lp in enumerate(params["layers"]):
        We1, be1 = lp["edge1"]["W"], lp["edge1"]["b"]
        C[f"L{l}_Whi"] = We1[0:_D]
        C[f"L{l}_Whj"] = We1[_D:2 * _D]
        C[f"L{l}_wd2"] = We1[2 * _D:2 * _D + 1]
        C[f"L{l}_wdl"] = We1[2 * _D + 1:2 * _D + 2]
        C[f"L{l}_geoC"] = struct @ We1[2 * _D + 2:] + be1
        C[f"L{l}_We2"] = lp["edge2"]["W"]
        C[f"L{l}_be2"] = lp["edge2"]["b"].reshape(1, -1)
        Wg1, bg1 = lp["gate1"]["W"], lp["gate1"]["b"]
        C[f"L{l}_wgd2"] = Wg1[0:1]
        C[f"L{l}_wgdl"] = Wg1[1:2]
        C[f"L{l}_gateC"] = struct @ Wg1[2:] + bg1
        C[f"L{l}_wg2r"] = lp["gate2"]["W"].reshape(1, -1)
        C[f"L{l}_bg2"] = lp["gate2"]["b"].reshape(1, 1)
        Wn1 = lp["node1"]["W"]
        C[f"L{l}_Wn1h"] = Wn1[0:_D]
        C[f"L{l}_Wn1a"] = Wn1[_D:]
        C[f"L{l}_bn1"] = lp["node1"]["b"].reshape(1, -1)
        C[f"L{l}_Wn2"] = lp["node2"]["W"]
        C[f"L{l}_bn2"] = lp["node2"]["b"].reshape(1, -1)
        C[f"L{l}_lng"] = lp["ln"]["g"].reshape(1, -1)
        C[f"L{l}_lnb"] = lp["ln"]["b"].reshape(1, -1)
    pma = params["pma"]
    qc = (pma["seed"].reshape(1, _D) @ pma["Wq"]["W"]
          + pma["Wq"]["b"]) / jnp.sqrt(jnp.float32(_DH))
    C["qc"] = qc
    head = jnp.arange(_D) // _DH
    Mh = (head[:, None] == jnp.arange(_NH)).astype(_F32)
    C["Mh"], C["MhT"] = Mh, Mh.T
    C["Wk"] = pma["Wk"]["W"]
    C["bk"] = pma["Wk"]["b"].reshape(1, -1)
    C["Wv"] = pma["Wv"]["W"]
    C["bv"] = pma["Wv"]["b"].reshape(1, -1)
    C["Wo"] = pma["Wo"]["W"]
    C["bo"] = pma["Wo"]["b"].reshape(1, -1)
    C["plng"] = pma["ln"]["g"].reshape(1, -1)
    C["plnb"] = pma["ln"]["b"].reshape(1, -1)
    C["Wo1"] = params["out1"]["W"]
    C["bo1"] = params["out1"]["b"].reshape(1, -1)
    C["Wo2"] = params["out2"]["W"]
    C["bo2"] = params["out2"]["b"].reshape(1, -1)
    return fo, jo, C


def kernel(xyz, finger_ids, joint_type_ids, edge_index, edge_type,
           edge_rest_lengths, params):
    fo, jo, C = _prepare(xyz, finger_ids, joint_type_ids, edge_index,
                         edge_type, edge_rest_lengths, params)
    xyznm = (xyz.reshape(_STEPS, _BB, _N, 3).transpose(0, 2, 1, 3)
             .reshape(_STEPS, _N, _BB * 3))
    return _forward(xyz, xyznm, fo, jo, C)


# speedup vs baseline: 1.0536x; 1.0536x over previous
import functools

import jax
import jax.numpy as jnp
from jax.experimental import pallas as pl
from jax.experimental.pallas import tpu as pltpu

_B, _N, _E = 256, 64, 256
_D, _DE, _DS, _NF, _NH, _DH, _OUT = 128, 64, 8, 10, 4, 32, 512
_BB = 16
_STEPS = _B // _BB
_F32 = jnp.float32


def _silu(x):
    return x * jax.nn.sigmoid(x)


def _ln(x, g, b):
    m = jnp.mean(x, axis=-1, keepdims=True)
    v = jnp.mean((x - m) ** 2, axis=-1, keepdims=True)
    return (x - m) / jnp.sqrt(v + 1e-5) * g + b


def _dot(a, b):
    return jnp.dot(a, b, preferred_element_type=_F32)


def _doth(a, b):
    return jnp.dot(a, b, preferred_element_type=_F32,
                   precision=jax.lax.Precision.HIGHEST)


def _forward(xyz, xyznm, fo, jo, consts, interpret=False):
    names = list(consts)

    def body(xyz_r, xyznm_r, fo_r, jo_r, *refs):
        cr = dict(zip(names, refs[: len(names)]))
        out_r = refs[len(names)]

        def g(n):
            return cr[n][...]

        Gij, GiT, Gd = g("Gij"), g("GiT"), g("Gd")
        rest = g("rest")

        diff = _dot(Gd, xyznm_r[0])
        d2A = _dot(diff * diff, g("S3"))
        distA = jnp.sqrt(d2A + 1e-9)
        dlA = (distA - rest) / (rest + 1e-9)
        d2cols = [d2A[:, b:b + 1] for b in range(_BB)]
        dlcols = [dlA[:, b:b + 1] for b in range(_BB)]

        xyzf = xyz_r[...].reshape(_BB * _N, 3)
        ff = fo_r[...].reshape(_BB * _N, 5)
        jj = jo_r[...].reshape(_BB * _N, 8)
        xb = _doth(xyzf, g("P"))
        h = (_doth(xyzf, g("Wx")) + _dot(jnp.sin(xb), g("Ws"))
             + _dot(jnp.cos(xb), g("Wc")) + _doth(ff, g("Wf"))
             + _doth(jj, g("Wjn")) + g("b1"))
        h = _ln(h, g("ln1g"), g("ln1b"))
        h = 0.5 * h * (1.0 + jax.lax.erf(h * (2.0 ** -0.5)))
        h = _dot(h, g("W2")) + g("b2")
        H = _ln(h, g("ln2g"), g("ln2b"))

        for l in range(3):
            p = lambda n: g(f"L{l}_{n}")
            HA = _dot(H, p("Whi"))
            HB = _dot(H, p("Whj"))
            m1s, ggs = [], []
            for b in range(_BB):
                hAB = jnp.concatenate(
                    [HA[b * _N:(b + 1) * _N], HB[b * _N:(b + 1) * _N]],
                    axis=0)
                m1s.append(_dot(Gij, hAB)
                           + d2cols[b] * p("wd2") + dlcols[b] * p("wdl")
                           + p("geoC"))
                gt = _silu(d2cols[b] * p("wgd2") + dlcols[b] * p("wgdl")
                           + p("gateC"))
                ggs.append(jax.nn.sigmoid(
                    jnp.sum(gt * p("wg2r"), axis=-1, keepdims=True)
                    + p("bg2")))
            m1 = jnp.concatenate(m1s, axis=0)
            m2 = _silu(_dot(_silu(m1), p("We2")) + p("be2"))
            aggs = [_dot(GiT, m2[b * _E:(b + 1) * _E] * ggs[b])
                    for b in range(_BB)]
            agg = jnp.concatenate(aggs, axis=0)
            upd = _silu(_dot(H, p("Wn1h")) + _dot(agg, p("Wn1a")) + p("bn1"))
            upd = _dot(upd, p("Wn2")) + p("bn2")
            H = _ln(H + upd, p("lng"), p("lnb"))

        q = g("qc")
        k = _dot(H, g("Wk")) + g("bk")
        v = _dot(H, g("Wv")) + g("bv")
        sb = _dot(k * q, g("Mh")).reshape(_BB, _N, _NH)
        mx = jnp.max(sb, axis=1, keepdims=True)
        e = jnp.exp(sb - mx)
        att = (e / jnp.sum(e, axis=1, keepdims=True)).reshape(_BB * _N, _NH)
        ae = _dot(att, g("MhT"))
        z = jnp.sum((ae * v).reshape(_BB, _N, _D), axis=1)
        z = _doth(z, g("Wo")) + g("bo")
        z = _ln(z, g("plng"), g("plnb"))
        y = _silu(_doth(z, g("Wo1")) + g("bo1"))
        out_r[...] = _doth(y, g("Wo2")) + g("bo2")

    def cmap(r):
        return lambda b: (0,) * r

    in_specs = [
        pl.BlockSpec((_BB, _N, 3), lambda b: (b, 0, 0)),
        pl.BlockSpec((1, _N, _BB * 3), lambda b: (b, 0, 0)),
        pl.BlockSpec((_BB, _N, 5), lambda b: (b, 0, 0)),
        pl.BlockSpec((_BB, _N, 8), lambda b: (b, 0, 0)),
    ] + [pl.BlockSpec(consts[n].shape, cmap(consts[n].ndim)) for n in names]

    return pl.pallas_call(
        body,
        grid=(_STEPS,),
        in_specs=in_specs,
        out_specs=pl.BlockSpec((_BB, _OUT), lambda b: (b, 0)),
        out_shape=jax.ShapeDtypeStruct((_B, _OUT), _F32),
        compiler_params=pltpu.CompilerParams(
            dimension_semantics=("parallel",)),
        interpret=interpret,
    )(xyz, xyznm, fo, jo, *consts.values())


def _prepare(xyz, finger_ids, joint_type_ids, edge_index, edge_type,
             edge_rest_lengths, params):
    i, j = edge_index[0], edge_index[1]
    an = jnp.arange(_N)
    Gi = (i[:, None] == an).astype(_F32)
    Gj = (j[:, None] == an).astype(_F32)
    fo = (finger_ids[..., None] == jnp.arange(5)).astype(_F32)
    jo = (joint_type_ids[..., None] == jnp.arange(8)).astype(_F32)
    struct = (edge_type[:, None] == jnp.arange(4)).astype(_F32) \
        @ params["edge_struct_emb"]

    C = {}
    C["Gij"] = jnp.concatenate([Gi, Gj], axis=1)
    C["GiT"], C["Gd"] = Gi.T, Gi - Gj
    C["rest"] = edge_rest_lengths.reshape(_E, 1)
    bidx = jnp.arange(_BB * 3) // 3
    C["S3"] = (bidx[:, None] == jnp.arange(_BB)).astype(_F32)
    P = jnp.zeros((3, 3 * _NF), _F32)
    freqs = 2.0 ** jnp.arange(_NF, dtype=_F32)
    for c in range(3):
        P = P.at[c, c * _NF:(c + 1) * _NF].set(freqs)
    C["P"] = P
    W1, b1 = params["proj1"]["W"], params["proj1"]["b"]
    C["Wx"], C["Ws"], C["Wc"] = W1[0:3], W1[3:33], W1[33:63]
    C["Wf"] = params["finger_emb"] @ W1[63:79]
    C["Wjn"] = params["joint_emb"] @ W1[79:95]
    C["b1"] = b1.reshape(1, -1)
    C["ln1g"] = params["proj_ln1"]["g"].reshape(1, -1)
    C["ln1b"] = params["proj_ln1"]["b"].reshape(1, -1)
    C["W2"] = params["proj2"]["W"]
    C["b2"] = params["proj2"]["b"].reshape(1, -1)
    C["ln2g"] = params["proj_ln2"]["g"].reshape(1, -1)
    C["ln2b"] = params["proj_ln2"]["b"].reshape(1, -1)
    for l, lp in enumerate(params["layers"]):
        We1, be1 = lp["edge1"]["W"], lp["edge1"]["b"]
        C[f"L{l}_Whi"] = We1[0:_D]
        C[f"L{l}_Whj"] = We1[_D:2 * _D]
        C[f"L{l}_wd2"] = We1[2 * _D:2 * _D + 1]
        C[f"L{l}_wdl"] = We1[2 * _D + 1:2 * _D + 2]
        C[f"L{l}_geoC"] = struct @ We1[2 * _D + 2:] + be1
        C[f"L{l}_We2"] = lp["edge2"]["W"]
        C[f"L{l}_be2"] = lp["edge2"]["b"].reshape(1, -1)
        Wg1, bg1 = lp["gate1"]["W"], lp["gate1"]["b"]
        C[f"L{l}_wgd2"] = Wg1[0:1]
        C[f"L{l}_wgdl"] = Wg1[1:2]
        C[f"L{l}_gateC"] = struct @ Wg1[2:] + bg1
        C[f"L{l}_wg2r"] = lp["gate2"]["W"].reshape(1, -1)
        C[f"L{l}_bg2"] = lp["gate2"]["b"].reshape(1, 1)
        Wn1 = lp["node1"]["W"]
        C[f"L{l}_Wn1h"] = Wn1[0:_D]
        C[f"L{l}_Wn1a"] = Wn1[_D:]
        C[f"L{l}_bn1"] = lp["node1"]["b"].reshape(1, -1)
        C[f"L{l}_Wn2"] = lp["node2"]["W"]
        C[f"L{l}_bn2"] = lp["node2"]["b"].reshape(1, -1)
        C[f"L{l}_lng"] = lp["ln"]["g"].reshape(1, -1)
        C[f"L{l}_lnb"] = lp["ln"]["b"].reshape(1, -1)
    pma = params["pma"]
    qc = (pma["seed"].reshape(1, _D) @ pma["Wq"]["W"]
          + pma["Wq"]["b"]) / jnp.sqrt(jnp.float32(_DH))
    C["qc"] = qc
    head = jnp.arange(_D) // _DH
    Mh = (head[:, None] == jnp.arange(_NH)).astype(_F32)
    C["Mh"], C["MhT"] = Mh, Mh.T
    C["Wk"] = pma["Wk"]["W"]
    C["bk"] = pma["Wk"]["b"].reshape(1, -1)
    C["Wv"] = pma["Wv"]["W"]
    C["bv"] = pma["Wv"]["b"].reshape(1, -1)
    C["Wo"] = pma["Wo"]["W"]
    C["bo"] = pma["Wo"]["b"].reshape(1, -1)
    C["plng"] = pma["ln"]["g"].reshape(1, -1)
    C["plnb"] = pma["ln"]["b"].reshape(1, -1)
    C["Wo1"] = params["out1"]["W"]
    C["bo1"] = params["out1"]["b"].reshape(1, -1)
    C["Wo2"] = params["out2"]["W"]
    C["bo2"] = params["out2"]["b"].reshape(1, -1)
    return fo, jo, C


def kernel(xyz, finger_ids, joint_type_ids, edge_index, edge_type,
           edge_rest_lengths, params):
    fo, jo, C = _prepare(xyz, finger_ids, joint_type_ids, edge_index,
                         edge_type, edge_rest_lengths, params)
    xyznm = (xyz.reshape(_STEPS, _BB, _N, 3).transpose(0, 2, 1, 3)
             .reshape(_STEPS, _N, _BB * 3))
    return _forward(xyz, xyznm, fo, jo, C)


# speedup vs baseline: 1.0565x; 1.0027x over previous
import functools

import jax
import jax.numpy as jnp
from jax.experimental import pallas as pl
from jax.experimental.pallas import tpu as pltpu

_B, _N, _E = 256, 64, 256
_D, _DE, _DS, _NF, _NH, _DH, _OUT = 128, 64, 8, 10, 4, 32, 512
_BB = 16
_STEPS = _B // _BB
_F32 = jnp.float32


def _silu(x):
    return x * jax.nn.sigmoid(x)


def _ln(x, g, b):
    m = jnp.mean(x, axis=-1, keepdims=True)
    v = jnp.mean((x - m) ** 2, axis=-1, keepdims=True)
    return (x - m) / jnp.sqrt(v + 1e-5) * g + b


def _dot(a, b):
    return jnp.dot(a, b, preferred_element_type=_F32)


def _doth(a, b):
    return jnp.dot(a, b, preferred_element_type=_F32,
                   precision=jax.lax.Precision.HIGHEST)


def _forward(xyz, xyznm, fo, jo, consts, interpret=False):
    names = list(consts)

    def body(xyz_r, xyznm_r, fo_r, jo_r, *refs):
        cr = dict(zip(names, refs[: len(names)]))
        out_r = refs[len(names)]

        def g(n):
            return cr[n][...]

        Gij, GiT, Gd = g("Gij"), g("GiT"), g("Gd")
        rest = g("rest")

        diff = _dot(Gd, xyznm_r[0])
        d2A = _dot(diff * diff, g("S3"))
        distA = jnp.sqrt(d2A + 1e-9)
        dlA = (distA - rest) / (rest + 1e-9)
        t2 = jnp.transpose(d2A)[:, :, None]
        tl = jnp.transpose(dlA)[:, :, None]

        xyzf = xyz_r[...].reshape(_BB * _N, 3)
        ff = fo_r[...].reshape(_BB * _N, 5)
        jj = jo_r[...].reshape(_BB * _N, 8)
        xb = _doth(xyzf, g("P"))
        h = (_doth(xyzf, g("Wx")) + _dot(jnp.sin(xb), g("Ws"))
             + _dot(jnp.cos(xb), g("Wc")) + _doth(ff, g("Wf"))
             + _doth(jj, g("Wjn")) + g("b1"))
        h = _ln(h, g("ln1g"), g("ln1b"))
        h = 0.5 * h * (1.0 + jax.lax.erf(h * (2.0 ** -0.5)))
        h = _dot(h, g("W2")) + g("b2")
        H = _ln(h, g("ln2g"), g("ln2b"))

        for l in range(3):
            p = lambda n: g(f"L{l}_{n}")
            HA = _dot(H, p("Whi"))
            HB = _dot(H, p("Whj"))
            m1s = []
            for b in range(_BB):
                hAB = jnp.concatenate(
                    [HA[b * _N:(b + 1) * _N], HB[b * _N:(b + 1) * _N]],
                    axis=0)
                m1s.append(_dot(Gij, hAB))
            m1 = (jnp.concatenate(m1s, axis=0).reshape(_BB, _E, _DE)
                  + t2 * p("wd2")[None] + tl * p("wdl")[None]
                  + p("geoC")[None])
            m2 = _silu(_dot(_silu(m1.reshape(_BB * _E, _DE)), p("We2"))
                       + p("be2"))
            gt = _silu(t2 * p("wgd2")[None] + tl * p("wgdl")[None]
                       + p("gateC")[None])
            gg = jax.nn.sigmoid(
                jnp.sum(gt * p("wg2r")[None], axis=-1, keepdims=True)
                + p("bg2"))
            mg = (m2.reshape(_BB, _E, _DE) * gg).reshape(_BB * _E, _DE)
            aggs = [_dot(GiT, mg[b * _E:(b + 1) * _E]) for b in range(_BB)]
            agg = jnp.concatenate(aggs, axis=0)
            upd = _silu(_dot(H, p("Wn1h")) + _dot(agg, p("Wn1a")) + p("bn1"))
            upd = _dot(upd, p("Wn2")) + p("bn2")
            H = _ln(H + upd, p("lng"), p("lnb"))

        q = g("qc")
        k = _dot(H, g("Wk")) + g("bk")
        v = _dot(H, g("Wv")) + g("bv")
        sb = _dot(k * q, g("Mh")).reshape(_BB, _N, _NH)
        mx = jnp.max(sb, axis=1, keepdims=True)
        e = jnp.exp(sb - mx)
        att = (e / jnp.sum(e, axis=1, keepdims=True)).reshape(_BB * _N, _NH)
        ae = _dot(att, g("MhT"))
        z = jnp.sum((ae * v).reshape(_BB, _N, _D), axis=1)
        z = _doth(z, g("Wo")) + g("bo")
        z = _ln(z, g("plng"), g("plnb"))
        y = _silu(_doth(z, g("Wo1")) + g("bo1"))
        out_r[...] = _doth(y, g("Wo2")) + g("bo2")

    def cmap(r):
        return lambda b: (0,) * r

    in_specs = [
        pl.BlockSpec((_BB, _N, 3), lambda b: (b, 0, 0)),
        pl.BlockSpec((1, _N, _BB * 3), lambda b: (b, 0, 0)),
        pl.BlockSpec((_BB, _N, 5), lambda b: (b, 0, 0)),
        pl.BlockSpec((_BB, _N, 8), lambda b: (b, 0, 0)),
    ] + [pl.BlockSpec(consts[n].shape, cmap(consts[n].ndim)) for n in names]

    return pl.pallas_call(
        body,
        grid=(_STEPS,),
        in_specs=in_specs,
        out_specs=pl.BlockSpec((_BB, _OUT), lambda b: (b, 0)),
        out_shape=jax.ShapeDtypeStruct((_B, _OUT), _F32),
        compiler_params=pltpu.CompilerParams(
            dimension_semantics=("parallel",)),
        interpret=interpret,
    )(xyz, xyznm, fo, jo, *consts.values())


def _prepare(xyz, finger_ids, joint_type_ids, edge_index, edge_type,
             edge_rest_lengths, params):
    i, j = edge_index[0], edge_index[1]
    an = jnp.arange(_N)
    Gi = (i[:, None] == an).astype(_F32)
    Gj = (j[:, None] == an).astype(_F32)
    fo = (finger_ids[..., None] == jnp.arange(5)).astype(_F32)
    jo = (joint_type_ids[..., None] == jnp.arange(8)).astype(_F32)
    struct = (edge_type[:, None] == jnp.arange(4)).astype(_F32) \
        @ params["edge_struct_emb"]

    C = {}
    C["Gij"] = jnp.concatenate([Gi, Gj], axis=1)
    C["GiT"], C["Gd"] = Gi.T, Gi - Gj
    C["rest"] = edge_rest_lengths.reshape(_E, 1)
    bidx = jnp.arange(_BB * 3) // 3
    C["S3"] = (bidx[:, None] == jnp.arange(_BB)).astype(_F32)
    P = jnp.zeros((3, 3 * _NF), _F32)
    freqs = 2.0 ** jnp.arange(_NF, dtype=_F32)
    for c in range(3):
        P = P.at[c, c * _NF:(c + 1) * _NF].set(freqs)
    C["P"] = P
    W1, b1 = params["proj1"]["W"], params["proj1"]["b"]
    C["Wx"], C["Ws"], C["Wc"] = W1[0:3], W1[3:33], W1[33:63]
    C["Wf"] = params["finger_emb"] @ W1[63:79]
    C["Wjn"] = params["joint_emb"] @ W1[79:95]
    C["b1"] = b1.reshape(1, -1)
    C["ln1g"] = params["proj_ln1"]["g"].reshape(1, -1)
    C["ln1b"] = params["proj_ln1"]["b"].reshape(1, -1)
    C["W2"] = params["proj2"]["W"]
    C["b2"] = params["proj2"]["b"].reshape(1, -1)
    C["ln2g"] = params["proj_ln2"]["g"].reshape(1, -1)
    C["ln2b"] = params["proj_ln2"]["b"].reshape(1, -1)
    for l, lp in enumerate(params["layers"]):
        We1, be1 = lp["edge1"]["W"], lp["edge1"]["b"]
        C[f"L{l}_Whi"] = We1[0:_D]
        C[f"L{l}_Whj"] = We1[_D:2 * _D]
        C[f"L{l}_wd2"] = We1[2 * _D:2 * _D + 1]
        C[f"L{l}_wdl"] = We1[2 * _D + 1:2 * _D + 2]
        C[f"L{l}_geoC"] = struct @ We1[2 * _D + 2:] + be1
        C[f"L{l}_We2"] = lp["edge2"]["W"]
        C[f"L{l}_be2"] = lp["edge2"]["b"].reshape(1, -1)
        Wg1, bg1 = lp["gate1"]["W"], lp["gate1"]["b"]
        C[f"L{l}_wgd2"] = Wg1[0:1]
        C[f"L{l}_wgdl"] = Wg1[1:2]
        C[f"L{l}_gateC"] = struct @ Wg1[2:] + bg1
        C[f"L{l}_wg2r"] = lp["gate2"]["W"].reshape(1, -1)
        C[f"L{l}_bg2"] = lp["gate2"]["b"].reshape(1, 1)
        Wn1 = lp["node1"]["W"]
        C[f"L{l}_Wn1h"] = Wn1[0:_D]
        C[f"L{l}_Wn1a"] = Wn1[_D:]
        C[f"L{l}_bn1"] = lp["node1"]["b"].reshape(1, -1)
        C[f"L{l}_Wn2"] = lp["node2"]["W"]
        C[f"L{l}_bn2"] = lp["node2"]["b"].reshape(1, -1)
        C[f"L{l}_lng"] = lp["ln"]["g"].reshape(1, -1)
        C[f"L{l}_lnb"] = lp["ln"]["b"].reshape(1, -1)
    pma = params["pma"]
    qc = (pma["seed"].reshape(1, _D) @ pma["Wq"]["W"]
          + pma["Wq"]["b"]) / jnp.sqrt(jnp.float32(_DH))
    C["qc"] = qc
    head = jnp.arange(_D) // _DH
    Mh = (head[:, None] == jnp.arange(_NH)).astype(_F32)
    C["Mh"], C["MhT"] = Mh, Mh.T
    C["Wk"] = pma["Wk"]["W"]
    C["bk"] = pma["Wk"]["b"].reshape(1, -1)
    C["Wv"] = pma["Wv"]["W"]
    C["bv"] = pma["Wv"]["b"].reshape(1, -1)
    C["Wo"] = pma["Wo"]["W"]
    C["bo"] = pma["Wo"]["b"].reshape(1, -1)
    C["plng"] = pma["ln"]["g"].reshape(1, -1)
    C["plnb"] = pma["ln"]["b"].reshape(1, -1)
    C["Wo1"] = params["out1"]["W"]
    C["bo1"] = params["out1"]["b"].reshape(1, -1)
    C["Wo2"] = params["out2"]["W"]
    C["bo2"] = params["out2"]["b"].reshape(1, -1)
    return fo, jo, C


def kernel(xyz, finger_ids, joint_type_ids, edge_index, edge_type,
           edge_rest_lengths, params):
    fo, jo, C = _prepare(xyz, finger_ids, joint_type_ids, edge_index,
                         edge_type, edge_rest_lengths, params)
    xyznm = (xyz.reshape(_STEPS, _BB, _N, 3).transpose(0, 2, 1, 3)
             .reshape(_STEPS, _N, _BB * 3))
    return _forward(xyz, xyznm, fo, jo, C)
